# baseline (speedup 1.0000x reference)
import jax
import jax.numpy as jnp
from jax.experimental import pallas as pl
from jax.experimental.pallas import tpu as pltpu
from jax.experimental.pallas import tpu_sc as plsc

QUEUE_SIZE = 131072
FEATURE_DIM = 128
BATCH = 16384

BLOCK_ROWS = 8192
GRID = QUEUE_SIZE // BLOCK_ROWS
FEAT_BLOCKS = BATCH // BLOCK_ROWS

LBL_COLS = 128
LBL_ROWS_Q = QUEUE_SIZE // LBL_COLS
LBL_ROWS_B = BATCH // LBL_COLS
LBL_ROWS_T = LBL_ROWS_Q - LBL_ROWS_B
LBL_BLOCK = 32
HEAD_BLOCKS = LBL_ROWS_B // LBL_BLOCK
TAIL_BLOCKS = LBL_ROWS_T // LBL_BLOCK


def _tc_copy_kernel(feat, features, out_f):
    i = pl.program_id(0)

    @pl.when(i < FEAT_BLOCKS)
    def _():
        out_f[...] = feat[...]

    @pl.when(i >= FEAT_BLOCKS)
    def _():
        out_f[...] = features[...]


def _features_copy(feat, features):
    new_idx = lambda i: (jnp.minimum(i, FEAT_BLOCKS - 1), 0)
    old_idx = lambda i: (jnp.maximum(i, FEAT_BLOCKS), 0)
    return pl.pallas_call(
        _tc_copy_kernel,
        grid=(GRID,),
        in_specs=[
            pl.BlockSpec((BLOCK_ROWS, FEATURE_DIM), new_idx),
            pl.BlockSpec((BLOCK_ROWS, FEATURE_DIM), old_idx),
        ],
        out_specs=pl.BlockSpec((BLOCK_ROWS, FEATURE_DIM), lambda i: (i, 0)),
        out_shape=jax.ShapeDtypeStruct((QUEUE_SIZE, FEATURE_DIM), jnp.float32),
        compiler_params=pltpu.CompilerParams(
            dimension_semantics=("arbitrary",),
        ),
    )(feat, features)


def _labels_copy(true2d, pred2d, tl2d, pl2d):
    mesh = plsc.VectorSubcoreMesh(core_axis_name="c", subcore_axis_name="s")
    out_ty = jax.ShapeDtypeStruct((LBL_ROWS_Q, LBL_COLS), jnp.int32)

    @pl.kernel(out_type=(out_ty, out_ty), mesh=mesh, scratch_types=[])
    def sc_kernel(t_hbm, p_hbm, tl_hbm, pl_hbm, ot_hbm, op_hbm):
        def body(t_in, p_in, t_out, p_out):
            t_out[...] = t_in[...]
            p_out[...] = p_in[...]

        blk = lambda: pl.BlockSpec((LBL_BLOCK, LBL_COLS), lambda i: (i, 0))
        pltpu.emit_pipeline(
            body,
            grid=(HEAD_BLOCKS,),
            in_specs=[blk(), blk()],
            out_specs=[blk(), blk()],
            core_axis_name=("c", "s"),
            dimension_semantics=(pltpu.PARALLEL,),
        )(t_hbm, p_hbm, ot_hbm, op_hbm)

        tblk = lambda: pl.BlockSpec((LBL_BLOCK, LBL_COLS),
                                    lambda i: (i + HEAD_BLOCKS, 0))
        pltpu.emit_pipeline(
            body,
            grid=(TAIL_BLOCKS,),
            in_specs=[tblk(), tblk()],
            out_specs=[tblk(), tblk()],
            core_axis_name=("c", "s"),
            dimension_semantics=(pltpu.PARALLEL,),
        )(tl_hbm, pl_hbm, ot_hbm, op_hbm)

    return sc_kernel(true2d, pred2d, tl2d, pl2d)


def kernel(feat, true, pred, features, true_labels, pred_labels):
    true2d = true.reshape(LBL_ROWS_B, LBL_COLS)
    pred2d = pred.reshape(LBL_ROWS_B, LBL_COLS)
    tl2d = true_labels.reshape(LBL_ROWS_Q, LBL_COLS)
    pl2d = pred_labels.reshape(LBL_ROWS_Q, LBL_COLS)

    out_t, out_p = _labels_copy(true2d, pred2d, tl2d, pl2d)
    out_f = _features_copy(feat, features)

    return (out_f, out_t.reshape(QUEUE_SIZE), out_p.reshape(QUEUE_SIZE))


# speedup vs baseline: 1.3504x; 1.3504x over previous
import jax
import jax.numpy as jnp
from jax.experimental import pallas as pl
from jax.experimental.pallas import tpu as pltpu

QUEUE_SIZE = 131072
FEATURE_DIM = 128
BATCH = 16384

BLOCK_ROWS = 8192
GRID = QUEUE_SIZE // BLOCK_ROWS
FEAT_BLOCKS = BATCH // BLOCK_ROWS

LBL_COLS = 128
LBL_ROWS_Q = QUEUE_SIZE // LBL_COLS
LBL_ROWS_B = BATCH // LBL_COLS


def _copy_kernel(feat, true2d, pred2d, features, tl2d, pl2d,
                 out_f, out_t, out_p):
    i = pl.program_id(0)

    @pl.when(i < FEAT_BLOCKS)
    def _():
        out_f[...] = feat[...]

    @pl.when(i >= FEAT_BLOCKS)
    def _():
        out_f[...] = features[...]

    @pl.when(i == 0)
    def _():
        out_t[: LBL_ROWS_B] = true2d[...]
        out_t[LBL_ROWS_B :] = tl2d[LBL_ROWS_B :]
        out_p[: LBL_ROWS_B] = pred2d[...]
        out_p[LBL_ROWS_B :] = pl2d[LBL_ROWS_B :]


def kernel(feat, true, pred, features, true_labels, pred_labels):
    true2d = true.reshape(LBL_ROWS_B, LBL_COLS)
    pred2d = pred.reshape(LBL_ROWS_B, LBL_COLS)
    tl2d = true_labels.reshape(LBL_ROWS_Q, LBL_COLS)
    pl2d = pred_labels.reshape(LBL_ROWS_Q, LBL_COLS)

    new_idx = lambda i: (jnp.minimum(i, FEAT_BLOCKS - 1), 0)
    old_idx = lambda i: (jnp.maximum(i, FEAT_BLOCKS), 0)
    whole = lambda i: (0, 0)

    out_f, out_t, out_p = pl.pallas_call(
        _copy_kernel,
        grid=(GRID,),
        in_specs=[
            pl.BlockSpec((BLOCK_ROWS, FEATURE_DIM), new_idx),
            pl.BlockSpec((LBL_ROWS_B, LBL_COLS), whole),
            pl.BlockSpec((LBL_ROWS_B, LBL_COLS), whole),
            pl.BlockSpec((BLOCK_ROWS, FEATURE_DIM), old_idx),
            pl.BlockSpec((LBL_ROWS_Q, LBL_COLS), whole),
            pl.BlockSpec((LBL_ROWS_Q, LBL_COLS), whole),
        ],
        out_specs=[
            pl.BlockSpec((BLOCK_ROWS, FEATURE_DIM), lambda i: (i, 0)),
            pl.BlockSpec((LBL_ROWS_Q, LBL_COLS), whole),
            pl.BlockSpec((LBL_ROWS_Q, LBL_COLS), whole),
        ],
        out_shape=[
            jax.ShapeDtypeStruct((QUEUE_SIZE, FEATURE_DIM), jnp.float32),
            jax.ShapeDtypeStruct((LBL_ROWS_Q, LBL_COLS), jnp.int32),
            jax.ShapeDtypeStruct((LBL_ROWS_Q, LBL_COLS), jnp.int32),
        ],
        compiler_params=pltpu.CompilerParams(
            dimension_semantics=("arbitrary",),
        ),
    )(feat, true2d, pred2d, features, tl2d, pl2d)

    return (out_f, out_t.reshape(QUEUE_SIZE), out_p.reshape(QUEUE_SIZE))
